# baseline (speedup 1.0000x reference)
import jax
import jax.numpy as jnp
from jax import lax
from jax.experimental import pallas as pl
from jax.experimental.pallas import tpu as pltpu
from jax.experimental.pallas import tpu_sc as plsc

F = 50000
V = 50000
E = 800000
S = 64

NC = 2
NS = 16
HSEG = 25000
ACC_ROWS = 25600
CH = 32
NCHUNK = E // CH
SENT = 1 << 20
L = 16
DUMP_A = 1568
DUMP_B = HSEG - 15 * DUMP_A


def _norm_body(x_ref, o_ref):
    x = x_ref[...]
    m = jnp.max(x, axis=1, keepdims=True)
    lse = m + jnp.log(jnp.sum(jnp.exp(x - m), axis=1, keepdims=True))
    o_ref[...] = x - lse


def _mlp_body(x_ref, w1_ref, b1_ref, w2_ref, b2_ref, o_ref):
    x = jnp.exp(x_ref[...])
    h = jnp.dot(x, w1_ref[...].T, preferred_element_type=jnp.float32) + b1_ref[...]
    h = jnp.maximum(h, 0.0)
    h = jnp.dot(h, w2_ref[...].T, preferred_element_type=jnp.float32) + b2_ref[...]
    h = jnp.maximum(h, 0.0) + 1e-19
    y = jnp.log(h)
    m = jnp.max(y, axis=1, keepdims=True)
    lse = m + jnp.log(jnp.sum(jnp.exp(y - m), axis=1, keepdims=True))
    o_ref[...] = y - lse


_TC_BLOCK = 2000


def _tc_normalize(x):
    n = x.shape[0]
    return pl.pallas_call(
        _norm_body,
        grid=(n // _TC_BLOCK,),
        in_specs=[pl.BlockSpec((_TC_BLOCK, S), lambda i: (i, 0))],
        out_specs=pl.BlockSpec((_TC_BLOCK, S), lambda i: (i, 0)),
        out_shape=jax.ShapeDtypeStruct((n, S), jnp.float32),
    )(x)


def _tc_mlp(x, W1, b1, W2, b2):
    n = x.shape[0]
    full = lambda i: (0, 0)
    return pl.pallas_call(
        _mlp_body,
        grid=(n // _TC_BLOCK,),
        in_specs=[
            pl.BlockSpec((_TC_BLOCK, S), lambda i: (i, 0)),
            pl.BlockSpec((S, S), full),
            pl.BlockSpec((1, S), full),
            pl.BlockSpec((S, S), full),
            pl.BlockSpec((1, S), full),
        ],
        out_specs=pl.BlockSpec((_TC_BLOCK, S), lambda i: (i, 0)),
        out_shape=jax.ShapeDtypeStruct((n, S), jnp.float32),
    )(x, W1, b1.reshape(1, S), W2, b2.reshape(1, S))


def _damp_body(a_ref, b_ref, c_ref, o_ref):
    o_ref[...] = 0.5 * (a_ref[...] - b_ref[...] + c_ref[...])


_E_BLOCK = 8000


def _tc_damp(a, b, c):
    spec = pl.BlockSpec((_E_BLOCK, S), lambda i: (i, 0))
    return pl.pallas_call(
        _damp_body,
        grid=(E // _E_BLOCK,),
        in_specs=[spec, spec, spec],
        out_specs=spec,
        out_shape=jax.ShapeDtypeStruct((E, S), jnp.float32),
    )(a, b, c)


def kernel(facToVar_edge_idx, prv_varToFactor_messages, prv_factorToVar_messages,
           prv_factor_beliefs, W1, b1, W2, b2):
    edge_f = facToVar_edge_idx[0]
    edge_v = facToVar_edge_idx[1]
    mapped_f = jnp.take(prv_factor_beliefs, edge_f, axis=0)
    ftv = _tc_damp(mapped_f, prv_varToFactor_messages, prv_factorToVar_messages)
    vb_raw = jax.ops.segment_sum(ftv, edge_v, num_segments=V)
    vb = _tc_normalize(vb_raw)
    mapped_v = jnp.take(vb, edge_v, axis=0)
    vtf = _tc_damp(mapped_v, ftv, prv_varToFactor_messages)
    nfb_raw = jax.ops.segment_sum(vtf, edge_f, num_segments=F)
    return _tc_mlp(nfb_raw, W1, b1, W2, b2)
